# baseline (speedup 1.0000x reference)
import functools
import math

import jax
import jax.numpy as jnp
from jax.experimental import pallas as pl
from jax.experimental.pallas import tpu as pltpu

N_EXP = 8
TOP_K = 1
CAPACITY_FACTOR = 1.0
MIN_CAPACITY = 4

BT = 128


def _capacity(num_tokens: int) -> int:
    capacity = math.floor(TOP_K * CAPACITY_FACTOR * num_tokens / N_EXP)
    capacity += capacity % 2
    return int(max(capacity, MIN_CAPACITY))


def _router_body(capacity, x_ref, wg_ref, uc_ref, cb_ref, sec_ref,
                 counts_ref):
    i = pl.program_id(0)

    @pl.when(i == 0)
    def _init():
        counts_ref[...] = jnp.zeros_like(counts_ref)

    logits = jax.lax.dot_general(
        x_ref[...], wg_ref[...],
        dimension_numbers=(((1,), (1,)), ((), ())),
        preferred_element_type=jnp.float32,
    )

    m = jnp.max(logits, axis=1, keepdims=True)
    eidx = jax.lax.broadcasted_iota(jnp.int32, (BT, N_EXP), 1)
    e = jnp.min(jnp.where(logits >= m, eidx, N_EXP), axis=1, keepdims=True)
    oh = (eidx == e).astype(jnp.float32)

    ri = jax.lax.broadcasted_iota(jnp.int32, (BT, BT), 0)
    ci = jax.lax.broadcasted_iota(jnp.int32, (BT, BT), 1)
    ltri = (ci < ri).astype(jnp.float32)
    prior = jax.lax.dot_general(
        ltri, oh, dimension_numbers=(((1,), (0,)), ((), ())),
        preferred_element_type=jnp.float32,
    )

    base = counts_ref[...]
    slot_all = prior.astype(jnp.int32) + base
    slots = jnp.sum(jnp.where(eidx == e, slot_all, 0), axis=1, keepdims=True)

    new_counts = base + jnp.sum(oh, axis=0, keepdims=True).astype(jnp.int32)
    counts_ref[...] = new_counts
    uc_ref[...] = jnp.minimum(new_counts, capacity)

    e3 = jax.lax.broadcasted_iota(jnp.int32, (BT, N_EXP, capacity), 1)
    c3 = jax.lax.broadcasted_iota(jnp.int32, (BT, N_EXP, capacity), 2)
    hit = (e3 == e[:, :, None]) & (c3 == slots[:, :, None])
    cb_ref[...] = hit.astype(jnp.float32)
    sec_ref[...] = hit


def kernel(x, w_g):
    num_tokens, n_embd = x.shape
    capacity = _capacity(num_tokens)
    grid = (num_tokens // BT,)
    body = functools.partial(_router_body, capacity)

    uc2, cb_weight, sec_mask = pl.pallas_call(
        body,
        grid=grid,
        in_specs=[
            pl.BlockSpec((BT, n_embd), lambda i: (i, 0)),
            pl.BlockSpec((N_EXP, n_embd), lambda i: (0, 0)),
        ],
        out_specs=[
            pl.BlockSpec((1, N_EXP), lambda i: (0, 0)),
            pl.BlockSpec((BT, N_EXP, capacity), lambda i: (i, 0, 0)),
            pl.BlockSpec((BT, N_EXP, capacity), lambda i: (i, 0, 0)),
        ],
        out_shape=[
            jax.ShapeDtypeStruct((1, N_EXP), jnp.int32),
            jax.ShapeDtypeStruct((num_tokens, N_EXP, capacity), jnp.float32),
            jax.ShapeDtypeStruct((num_tokens, N_EXP, capacity), jnp.bool_),
        ],
        scratch_shapes=[pltpu.VMEM((1, N_EXP), jnp.int32)],
    )(x, w_g)

    return uc2.reshape(N_EXP), cb_weight, sec_mask


# speedup vs baseline: 1.1348x; 1.1348x over previous
import functools
import math

import jax
import jax.numpy as jnp
from jax.experimental import pallas as pl
from jax.experimental.pallas import tpu as pltpu

N_EXP = 8
TOP_K = 1
CAPACITY_FACTOR = 1.0
MIN_CAPACITY = 4

BT = 512


def _capacity(num_tokens: int) -> int:
    capacity = math.floor(TOP_K * CAPACITY_FACTOR * num_tokens / N_EXP)
    capacity += capacity % 2
    return int(max(capacity, MIN_CAPACITY))


def _router_body(capacity, x_ref, wg_ref, uc_ref, cb_ref, sec_ref,
                 counts_ref):
    i = pl.program_id(0)

    @pl.when(i == 0)
    def _init():
        counts_ref[...] = jnp.zeros_like(counts_ref)

    logits = jax.lax.dot_general(
        x_ref[...], wg_ref[...],
        dimension_numbers=(((1,), (1,)), ((), ())),
        preferred_element_type=jnp.float32,
    )

    m = jnp.max(logits, axis=1, keepdims=True)
    eidx = jax.lax.broadcasted_iota(jnp.int32, (BT, N_EXP), 1)
    e = jnp.min(jnp.where(logits >= m, eidx, N_EXP), axis=1, keepdims=True)
    oh = (eidx == e).astype(jnp.float32)

    ri = jax.lax.broadcasted_iota(jnp.int32, (BT, BT), 0)
    ci = jax.lax.broadcasted_iota(jnp.int32, (BT, BT), 1)
    ltri = (ci < ri).astype(jnp.float32)
    prior = jax.lax.dot_general(
        ltri, oh, dimension_numbers=(((1,), (0,)), ((), ())),
        preferred_element_type=jnp.float32,
    )

    base = counts_ref[...]
    slot_all = prior.astype(jnp.int32) + base
    slots = jnp.sum(jnp.where(eidx == e, slot_all, 0), axis=1, keepdims=True)

    new_counts = base + jnp.sum(oh, axis=0, keepdims=True).astype(jnp.int32)
    counts_ref[...] = new_counts
    uc_ref[...] = jnp.minimum(new_counts, capacity)

    e3 = jax.lax.broadcasted_iota(jnp.int32, (BT, N_EXP, capacity), 1)
    c3 = jax.lax.broadcasted_iota(jnp.int32, (BT, N_EXP, capacity), 2)
    hit = (e3 == e[:, :, None]) & (c3 == slots[:, :, None])
    cb_ref[...] = hit.astype(jnp.float32)
    sec_ref[...] = hit


def kernel(x, w_g):
    num_tokens, n_embd = x.shape
    capacity = _capacity(num_tokens)
    grid = (num_tokens // BT,)
    body = functools.partial(_router_body, capacity)

    uc2, cb_weight, sec_mask = pl.pallas_call(
        body,
        grid=grid,
        in_specs=[
            pl.BlockSpec((BT, n_embd), lambda i: (i, 0)),
            pl.BlockSpec((N_EXP, n_embd), lambda i: (0, 0)),
        ],
        out_specs=[
            pl.BlockSpec((1, N_EXP), lambda i: (0, 0)),
            pl.BlockSpec((BT, N_EXP, capacity), lambda i: (i, 0, 0)),
            pl.BlockSpec((BT, N_EXP, capacity), lambda i: (i, 0, 0)),
        ],
        out_shape=[
            jax.ShapeDtypeStruct((1, N_EXP), jnp.int32),
            jax.ShapeDtypeStruct((num_tokens, N_EXP, capacity), jnp.float32),
            jax.ShapeDtypeStruct((num_tokens, N_EXP, capacity), jnp.bool_),
        ],
        scratch_shapes=[pltpu.VMEM((1, N_EXP), jnp.int32)],
    )(x, w_g)

    return uc2.reshape(N_EXP), cb_weight, sec_mask
